# baseline (speedup 1.0000x reference)
import functools

import jax
import jax.numpy as jnp
from jax.experimental import pallas as pl
from jax.experimental.pallas import tpu as pltpu

N_NODES = 10000
N_EDGES = 160000
N_GRAPHS = 16

_ROW_BLK = 1000


def _qkvs_body(nx, act, x_ref, w_ref, b_ref, o_ref):
    x = x_ref[...] if nx == 1 else jnp.concatenate([x_ref[0], x_ref[1]], axis=1)
    if act:
        x = jnp.where(x > 0, x, jnp.exp(jnp.minimum(x, 0.0)) - 1.0)
    o_ref[...] = (
        jnp.dot(x, w_ref[...], preferred_element_type=jnp.float32) + b_ref[...]
    )


def _qkvs_matmul(xs, w, b, act):
    nx = len(xs)
    n = xs[0].shape[0]
    din = sum(x.shape[1] for x in xs)
    dout4 = w.shape[1]
    grid = n // _ROW_BLK
    in_specs = [
        pl.BlockSpec((_ROW_BLK, x.shape[1]), lambda i: (i, 0)) for x in xs
    ]
    x_arg = xs[0] if nx == 1 else jnp.stack(xs, axis=0)
    if nx > 1:
        in_specs = [pl.BlockSpec((nx, _ROW_BLK, xs[0].shape[1]), lambda i: (0, i, 0))]
    return pl.pallas_call(
        functools.partial(_qkvs_body, nx, act),
        grid=(grid,),
        in_specs=in_specs
        + [
            pl.BlockSpec((din, dout4), lambda i: (0, 0)),
            pl.BlockSpec((1, dout4), lambda i: (0, 0)),
        ],
        out_specs=pl.BlockSpec((_ROW_BLK, dout4), lambda i: (i, 0)),
        out_shape=jax.ShapeDtypeStruct((n, dout4), jnp.float32),
    )(x_arg, w, b.reshape(1, -1))


def _layer(xs, p, edge_index, act):
    w = jnp.concatenate([p["Wq"], p["Wk"], p["Wv"], p["Ws"]], axis=1)
    b = jnp.concatenate([p["bq"], p["bk"], p["bv"], p["bs"]])
    d = p["Wq"].shape[1]
    qkvs = _qkvs_matmul(xs, w, b, act)
    q, k, v, s = (
        qkvs[:, :d],
        qkvs[:, d : 2 * d],
        qkvs[:, 2 * d : 3 * d],
        qkvs[:, 3 * d :],
    )
    src = edge_index[0]
    dst = edge_index[1]
    n = q.shape[0]
    alpha = jnp.sum(q[dst] * k[src], axis=-1) / jnp.sqrt(float(d))
    amax = jax.ops.segment_max(alpha, dst, num_segments=n)
    amax = jnp.where(jnp.isfinite(amax), amax, 0.0)
    e = jnp.exp(alpha - amax[dst])
    denom = jax.ops.segment_sum(e, dst, num_segments=n)
    wgt = e / (denom[dst] + 1e-16)
    agg = jax.ops.segment_sum(wgt[:, None] * v[src], dst, num_segments=n)
    return agg + s


def _gap(x, batch_index, num_graphs):
    counts = jax.ops.segment_sum(
        jnp.ones((x.shape[0],), x.dtype), batch_index, num_segments=num_graphs
    )
    sums = jax.ops.segment_sum(x, batch_index, num_segments=num_graphs)
    return sums / jnp.maximum(counts, 1.0)[:, None]


def kernel(features, img_feat, edge_index, batch_index, params):
    ei = edge_index
    h1 = _layer((features,), params["conv1"], ei, act=False)
    h2 = _layer((h1,), params["conv2"], ei, act=True)
    h3p = _layer((h2,), params["conv3"], ei, act=False)
    h4 = _layer((h3p,), params["conv4"], ei, act=True)
    img1p = _layer((img_feat,), params["imgconv1"], ei, act=False)
    img2 = _layer((img1p,), params["imgconv2"], ei, act=True)
    img3p = _layer((img2,), params["imgconv3"], ei, act=False)
    img4 = _layer((img3p,), params["imgconv4"], ei, act=True)
    combine_p = _layer((h2, img2), params["neck"], ei, act=False)
    c2 = _layer((combine_p,), params["neck2"], ei, act=True)
    c3p = _layer((c2,), params["c3"], ei, act=False)
    c4 = _layer((c3p,), params["c4"], ei, act=True)
    hidden = _gap(c2, batch_index, N_GRAPHS)
    return (h2, img2, c2, h4, img4, c4, hidden)


# speedup vs baseline: 1.7103x; 1.7103x over previous
import functools

import jax
import jax.numpy as jnp
from jax import lax
from jax.experimental import pallas as pl
from jax.experimental.pallas import tpu as pltpu
from jax.experimental.pallas import tpu_sc as plsc

N_NODES = 10000
N_EDGES = 160000
N_GRAPHS = 16

NN = 10240
EPAD = 163840
NW = 32
NS = 16
EP = EPAD // NW
CH = 128
NCH = EP // CH
NP = NN // NW
NEG = -3.0e38

_ROW_BLK = 640

_SC_PARAMS = pltpu.CompilerParams(needs_layout_passes=False)


def _qkvs_body(nx, act, d, nvb, *refs):
    x_refs = refs[:nx]
    w_ref, b_ref = refs[nx], refs[nx + 1]
    outs = refs[nx + 2:]
    x = x_refs[0][...] if nx == 1 else jnp.concatenate(
        [r[...] for r in x_refs], axis=1)
    if act:
        x = jnp.where(x > 0, x, jnp.exp(jnp.minimum(x, 0.0)) - 1.0)
    full = jnp.dot(x, w_ref[...], preferred_element_type=jnp.float32) + b_ref[...]
    outs[0][...] = full[:, :d]
    outs[1][...] = full[:, d:2 * d]
    outs[2][...] = full[:, 3 * d:]
    for i in range(nvb):
        outs[3 + i][...] = full[:, 2 * d + i * 128:2 * d + (i + 1) * 128]


def _qkvs_matmul(xs, w, b, act):
    nx = len(xs)
    din = sum(x.shape[1] for x in xs)
    d = w.shape[1] // 4
    nvb = d // 128
    grid = NN // _ROW_BLK
    in_specs = [pl.BlockSpec((_ROW_BLK, x.shape[1]), lambda i: (i, 0)) for x in xs]
    in_specs += [
        pl.BlockSpec((din, 4 * d), lambda i: (0, 0)),
        pl.BlockSpec((1, 4 * d), lambda i: (0, 0)),
    ]
    out_specs = [pl.BlockSpec((_ROW_BLK, d), lambda i: (i, 0))] * 3 + [
        pl.BlockSpec((_ROW_BLK, 128), lambda i: (i, 0))] * nvb
    out_shape = [jax.ShapeDtypeStruct((NN, d), jnp.float32)] * 3 + [
        jax.ShapeDtypeStruct((NN, 128), jnp.float32)] * nvb
    return pl.pallas_call(
        functools.partial(_qkvs_body, nx, act, d, nvb),
        grid=(grid,),
        in_specs=in_specs,
        out_specs=out_specs,
        out_shape=out_shape,
    )(*xs, w, b.reshape(1, -1))


def _combine_body(nvb, *refs):
    s_ref = refs[0]
    aggs = refs[1:1 + nvb]
    o_ref = refs[1 + nvb]
    out = s_ref[...]
    parts = []
    for i in range(nvb):
        parts.append(aggs[i][0] + aggs[i][1])
    o_ref[...] = out + jnp.concatenate(parts, axis=1)


def _combine(skip, agg_parts):
    d = skip.shape[1]
    nvb = d // 128
    grid = NN // _ROW_BLK
    in_specs = [pl.BlockSpec((_ROW_BLK, d), lambda i: (i, 0))] + [
        pl.BlockSpec((2, _ROW_BLK, 128), lambda i: (0, i, 0))] * nvb
    return pl.pallas_call(
        functools.partial(_combine_body, nvb),
        grid=(grid,),
        in_specs=in_specs,
        out_specs=pl.BlockSpec((_ROW_BLK, d), lambda i: (i, 0)),
        out_shape=jax.ShapeDtypeStruct((NN, d), jnp.float32),
    )(skip, *agg_parts)


def _gap_body(c2_ref, b_ref, o_ref):
    gids = lax.broadcasted_iota(jnp.int32, (N_GRAPHS, NN), 0)
    mask = (b_ref[...] == gids).astype(jnp.float32)
    sums = jnp.dot(mask, c2_ref[...], preferred_element_type=jnp.float32)
    counts = jnp.sum(mask, axis=1, keepdims=True)
    o_ref[...] = sums / jnp.maximum(counts, 1.0)


def _gap(c2_pad, batch_pad):
    return pl.pallas_call(
        _gap_body,
        in_specs=[
            pl.BlockSpec((NN, c2_pad.shape[1]), lambda: (0, 0)),
            pl.BlockSpec((1, NN), lambda: (0, 0)),
        ],
        out_specs=pl.BlockSpec((N_GRAPHS, c2_pad.shape[1]), lambda: (0, 0)),
        out_shape=jax.ShapeDtypeStruct((N_GRAPHS, c2_pad.shape[1]), jnp.float32),
    )(c2_pad, batch_pad.reshape(1, -1))


_GDN = lax.GatherDimensionNumbers(
    offset_dims=(), collapsed_slice_dims=(0,), start_index_map=(0,))


def _lane_shift(x, s):
    idx = (lax.iota(jnp.int32, 16) + s) & 15
    return lax.gather(x, idx[:, None], dimension_numbers=_GDN,
                      slice_sizes=(1,),
                      mode=lax.GatherScatterMode.PROMISE_IN_BOUNDS)


def _exp0(x):
    return jnp.exp(jnp.maximum(x, -87.0))


def _butterfly_softmax(key, m, s):
    for sh in (1, 2, 4, 8):
        k2 = _lane_shift(key, sh)
        m2 = jnp.where(key == k2, _lane_shift(m, sh), NEG)
        s2 = jnp.where(key == k2, _lane_shift(s, sh), 0.0)
        mm = jnp.maximum(m, m2)
        s = s * _exp0(m - mm) + s2 * _exp0(m2 - mm)
        m = mm
    return m, s


def _worker_id():
    return lax.axis_index("c") * NS + lax.axis_index("s")


def _alpha_body(d, q_hbm, k_hbm, src_hbm, dst_hbm,
                alpha_hbm, mpart_hbm, spart_hbm,
                srcv, dstv, qbuf, kbuf, abuf, mloc, sloc, sem1, sem2):
    w = _worker_id()
    scale = 1.0 / float(d) ** 0.5

    def init(i, _):
        mloc[pl.ds(i * 16, 16)] = jnp.full((16,), NEG, jnp.float32)
        sloc[pl.ds(i * 16, 16)] = jnp.zeros((16,), jnp.float32)
        return 0
    lax.fori_loop(0, NN // 16, init, 0)

    def chunk(ci, _):
        base = w * EP + ci * CH
        pltpu.sync_copy(src_hbm.at[pl.ds(base, CH)], srcv)
        pltpu.sync_copy(dst_hbm.at[pl.ds(base, CH)], dstv)
        cp1 = pltpu.async_copy(q_hbm.at[dstv], qbuf, sem1)
        cp2 = pltpu.async_copy(k_hbm.at[srcv], kbuf, sem2)
        cp1.wait()
        cp2.wait()

        def grp(g, _):
            lanes = lax.iota(jnp.int32, 16) + g * 16

            def dot(jo, acc):
                for u in range(16):
                    jv = jnp.full((16,), jo * 16 + u, jnp.int32)
                    acc = acc + (plsc.load_gather(qbuf, [lanes, jv]) *
                                 plsc.load_gather(kbuf, [lanes, jv]))
                return acc
            alpha = lax.fori_loop(0, d // 16, dot,
                                  jnp.zeros((16,), jnp.float32)) * scale
            dst16 = dstv[pl.ds(g * 16, 16)]
            m, ss = _butterfly_softmax(dst16, alpha,
                                       jnp.ones((16,), jnp.float32))
            curm = plsc.load_gather(mloc, [dst16])
            curs = plsc.load_gather(sloc, [dst16])
            mm = jnp.maximum(curm, m)
            snew = curs * _exp0(curm - mm) + ss * _exp0(m - mm)
            plsc.store_scatter(mloc, [dst16], mm)
            plsc.store_scatter(sloc, [dst16], snew)
            abuf[pl.ds(g * 16, 16)] = alpha
            return 0
        lax.fori_loop(0, CH // 16, grp, 0)
        pltpu.sync_copy(abuf, alpha_hbm.at[pl.ds(base, CH)])
        return 0
    lax.fori_loop(0, NCH, chunk, 0)
    pltpu.sync_copy(mloc, mpart_hbm.at[w])
    pltpu.sync_copy(sloc, spart_hbm.at[w])


def _alpha_kernel(d):
    mesh = plsc.VectorSubcoreMesh(core_axis_name="c", subcore_axis_name="s")
    return pl.kernel(
        functools.partial(_alpha_body, d),
        out_type=(
            jax.ShapeDtypeStruct((EPAD,), jnp.float32),
            jax.ShapeDtypeStruct((NW, NN), jnp.float32),
            jax.ShapeDtypeStruct((NW, NN), jnp.float32),
        ),
        mesh=mesh,
        scratch_types=[
            pltpu.VMEM((CH,), jnp.int32),
            pltpu.VMEM((CH,), jnp.int32),
            pltpu.VMEM((CH, d), jnp.float32),
            pltpu.VMEM((CH, d), jnp.float32),
            pltpu.VMEM((CH,), jnp.float32),
            pltpu.VMEM((NN,), jnp.float32),
            pltpu.VMEM((NN,), jnp.float32),
            pltpu.SemaphoreType.DMA,
            pltpu.SemaphoreType.DMA,
        ],
        compiler_params=_SC_PARAMS,
    )


def _merge_body(mpart_hbm, spart_hbm, mg_hbm, sg_hbm, blkm, blks, mgv, sgv):
    w = _worker_id()
    for t in range(NW):
        pltpu.sync_copy(mpart_hbm.at[pl.ds(t * NN + w * NP, NP)],
                        blkm.at[pl.ds(t * NP, NP)])
        pltpu.sync_copy(spart_hbm.at[pl.ds(t * NN + w * NP, NP)],
                        blks.at[pl.ds(t * NP, NP)])

    def col(i, _):
        m = jnp.full((16,), NEG, jnp.float32)
        for t in range(NW):
            m = jnp.maximum(m, blkm[pl.ds(t * NP + i * 16, 16)])
        s = jnp.zeros((16,), jnp.float32)
        for t in range(NW):
            mt = blkm[pl.ds(t * NP + i * 16, 16)]
            s = s + blks[pl.ds(t * NP + i * 16, 16)] * _exp0(mt - m)
        mgv[pl.ds(i * 16, 16)] = m
        sgv[pl.ds(i * 16, 16)] = s
        return 0
    lax.fori_loop(0, NP // 16, col, 0)
    pltpu.sync_copy(mgv, mg_hbm.at[pl.ds(w * NP, NP)])
    pltpu.sync_copy(sgv, sg_hbm.at[pl.ds(w * NP, NP)])


def _merge_kernel():
    mesh = plsc.VectorSubcoreMesh(core_axis_name="c", subcore_axis_name="s")
    return pl.kernel(
        _merge_body,
        out_type=(
            jax.ShapeDtypeStruct((NN,), jnp.float32),
            jax.ShapeDtypeStruct((NN,), jnp.float32),
        ),
        mesh=mesh,
        scratch_types=[
            pltpu.VMEM((NW * NP,), jnp.float32),
            pltpu.VMEM((NW * NP,), jnp.float32),
            pltpu.VMEM((NP,), jnp.float32),
            pltpu.VMEM((NP,), jnp.float32),
        ],
        compiler_params=_SC_PARAMS,
    )


def _agg_body(nvb, *refs):
    v_blocks = refs[:nvb]
    src_hbm, dst_hbm, alpha_hbm, mg_hbm, sg_hbm = refs[nvb:nvb + 5]
    agg_outs = refs[nvb + 5:nvb + 5 + nvb]
    (srcv, dstv, abuf, wbuf, vbuf, zbuf, mv, sv, aggsp, sem) = \
        refs[nvb + 5 + nvb:]
    cid = lax.axis_index("c")
    sid = lax.axis_index("s")
    w = cid * NS + sid

    pltpu.sync_copy(mg_hbm, mv)
    pltpu.sync_copy(sg_hbm, sv)

    def zrow(r, _):
        for jj in range(8):
            zbuf[r, pl.ds(jj * 16, 16)] = jnp.zeros((16,), jnp.float32)
        return 0
    lax.fori_loop(0, 8, zrow, 0)

    for blk in range(nvb):
        def zsp(i, _):
            pltpu.sync_copy(zbuf, aggsp.at[pl.ds(sid * (NN // NS) + i * 8, 8)])
            return 0
        lax.fori_loop(0, NN // NS // 8, zsp, 0)
        plsc.subcore_barrier()

        def chunk(ci, _):
            base = w * EP + ci * CH
            pltpu.sync_copy(src_hbm.at[pl.ds(base, CH)], srcv)
            pltpu.sync_copy(dst_hbm.at[pl.ds(base, CH)], dstv)
            pltpu.sync_copy(alpha_hbm.at[pl.ds(base, CH)], abuf)
            cp = pltpu.async_copy(v_blocks[blk].at[srcv], vbuf, sem)

            def grp(g, _):
                sl = pl.ds(g * 16, 16)
                dst16 = dstv[sl]
                a16 = abuf[sl]
                m16 = plsc.load_gather(mv, [dst16])
                s16 = plsc.load_gather(sv, [dst16])
                wbuf[sl] = _exp0(a16 - m16) / (s16 + 1e-16)
                return 0
            lax.fori_loop(0, CH // 16, grp, 0)
            cp.wait()

            def rowg(g, _):
                w16 = wbuf[pl.ds(g * 16, 16)]
                for u in range(16):
                    r = g * 16 + u
                    wv = jnp.full((16,), w16[u])
                    for jj in range(8):
                        sl = pl.ds(jj * 16, 16)
                        vbuf[r, sl] = vbuf[r, sl] * wv
                return 0
            lax.fori_loop(0, CH // 16, rowg, 0)
            pltpu.sync_copy(vbuf, aggsp.at[dstv], add=True)
            return 0
        lax.fori_loop(0, NCH, chunk, 0)
        plsc.subcore_barrier()

        def dump(i, _):
            rowbase = sid * (NN // NS) + i * 128
            pltpu.sync_copy(aggsp.at[pl.ds(rowbase, 128)],
                            agg_outs[blk].at[cid, pl.ds(rowbase, 128)])
            return 0
        lax.fori_loop(0, NN // NS // 128, dump, 0)
        plsc.subcore_barrier()


def _agg_kernel(d):
    nvb = d // 128
    mesh = plsc.VectorSubcoreMesh(core_axis_name="c", subcore_axis_name="s")
    return pl.kernel(
        functools.partial(_agg_body, nvb),
        out_type=tuple(
            jax.ShapeDtypeStruct((2, NN, 128), jnp.float32)
            for _ in range(nvb)),
        mesh=mesh,
        scratch_types=[
            pltpu.VMEM((CH,), jnp.int32),
            pltpu.VMEM((CH,), jnp.int32),
            pltpu.VMEM((CH,), jnp.float32),
            pltpu.VMEM((CH,), jnp.float32),
            pltpu.VMEM((CH, 128), jnp.float32),
            pltpu.VMEM((8, 128), jnp.float32),
            pltpu.VMEM((NN,), jnp.float32),
            pltpu.VMEM((NN,), jnp.float32),
            pltpu.VMEM_SHARED((NN, 128), jnp.float32),
            pltpu.SemaphoreType.DMA,
        ],
        compiler_params=_SC_PARAMS,
    )


def _layer(xs, p, src_p, dst_p, act):
    d = p["Wq"].shape[1]
    w = jnp.concatenate([p["Wq"], p["Wk"], p["Wv"], p["Ws"]], axis=1)
    b = jnp.concatenate([p["bq"], p["bk"], p["bv"], p["bs"]])
    outs = _qkvs_matmul(xs, w, b, act)
    q, k, skip = outs[0], outs[1], outs[2]
    v_blocks = outs[3:]
    alpha, mpart, spart = _alpha_kernel(d)(q, k, src_p, dst_p)
    mg, sg = _merge_kernel()(mpart.reshape(-1), spart.reshape(-1))
    agg_parts = _agg_kernel(d)(*v_blocks, src_p, dst_p, alpha, mg, sg)
    if not isinstance(agg_parts, (list, tuple)):
        agg_parts = (agg_parts,)
    return _combine(skip, agg_parts)


def kernel(features, img_feat, edge_index, batch_index, params):
    pad_n = NN - N_NODES
    feat_p = jnp.pad(features, ((0, pad_n), (0, 0)))
    img_p = jnp.pad(img_feat, ((0, pad_n), (0, 0)))
    src_p = jnp.pad(edge_index[0], (0, EPAD - N_EDGES))
    dst_p = jnp.pad(edge_index[1], (0, EPAD - N_EDGES),
                    constant_values=N_NODES)
    batch_p = jnp.pad(batch_index, (0, pad_n), constant_values=N_GRAPHS)

    h1 = _layer((feat_p,), params["conv1"], src_p, dst_p, act=False)
    h2 = _layer((h1,), params["conv2"], src_p, dst_p, act=True)
    h3p = _layer((h2,), params["conv3"], src_p, dst_p, act=False)
    h4 = _layer((h3p,), params["conv4"], src_p, dst_p, act=True)
    img1p = _layer((img_p,), params["imgconv1"], src_p, dst_p, act=False)
    img2 = _layer((img1p,), params["imgconv2"], src_p, dst_p, act=True)
    img3p = _layer((img2,), params["imgconv3"], src_p, dst_p, act=False)
    img4 = _layer((img3p,), params["imgconv4"], src_p, dst_p, act=True)
    combine_p = _layer((h2, img2), params["neck"], src_p, dst_p, act=False)
    c2 = _layer((combine_p,), params["neck2"], src_p, dst_p, act=True)
    c3p = _layer((c2,), params["c3"], src_p, dst_p, act=False)
    c4 = _layer((c3p,), params["c4"], src_p, dst_p, act=True)
    hidden = _gap(c2, batch_p)
    return (h2[:N_NODES], img2[:N_NODES], c2[:N_NODES], h4[:N_NODES],
            img4[:N_NODES], c4[:N_NODES], hidden)


# speedup vs baseline: 1.8713x; 1.0941x over previous
import functools

import jax
import jax.numpy as jnp
from jax import lax
from jax.experimental import pallas as pl
from jax.experimental.pallas import tpu as pltpu
from jax.experimental.pallas import tpu_sc as plsc

N_NODES = 10000
N_EDGES = 160000
N_GRAPHS = 16

NN = 10240
EPAD = 163840
NW = 32
NS = 16
EP = EPAD // NW
CH = 128
NCH = EP // CH
NP = NN // NW
NEG = -3.0e38

_ROW_BLK = 640

_SC_PARAMS = pltpu.CompilerParams(needs_layout_passes=False)


def _qkvs_body(nx, act, d, nvb, *refs):
    x_refs = refs[:nx]
    w_ref, b_ref = refs[nx], refs[nx + 1]
    outs = refs[nx + 2:]
    x = x_refs[0][...] if nx == 1 else jnp.concatenate(
        [r[...] for r in x_refs], axis=1)
    if act:
        x = jnp.where(x > 0, x, jnp.exp(jnp.minimum(x, 0.0)) - 1.0)
    full = jnp.dot(x, w_ref[...], preferred_element_type=jnp.float32) + b_ref[...]
    outs[0][...] = full[:, :d]
    outs[1][...] = full[:, d:2 * d]
    outs[2][...] = full[:, 3 * d:]
    for i in range(nvb):
        outs[3 + i][...] = full[:, 2 * d + i * 128:2 * d + (i + 1) * 128]


def _qkvs_matmul(xs, w, b, act):
    nx = len(xs)
    din = sum(x.shape[1] for x in xs)
    d = w.shape[1] // 4
    nvb = d // 128
    grid = NN // _ROW_BLK
    in_specs = [pl.BlockSpec((_ROW_BLK, x.shape[1]), lambda i: (i, 0)) for x in xs]
    in_specs += [
        pl.BlockSpec((din, 4 * d), lambda i: (0, 0)),
        pl.BlockSpec((1, 4 * d), lambda i: (0, 0)),
    ]
    out_specs = [pl.BlockSpec((_ROW_BLK, d), lambda i: (i, 0))] * 3 + [
        pl.BlockSpec((_ROW_BLK, 128), lambda i: (i, 0))] * nvb
    out_shape = [jax.ShapeDtypeStruct((NN, d), jnp.float32)] * 3 + [
        jax.ShapeDtypeStruct((NN, 128), jnp.float32)] * nvb
    return pl.pallas_call(
        functools.partial(_qkvs_body, nx, act, d, nvb),
        grid=(grid,),
        in_specs=in_specs,
        out_specs=out_specs,
        out_shape=out_shape,
    )(*xs, w, b.reshape(1, -1))


def _combine_body(nvb, *refs):
    s_ref = refs[0]
    aggs = refs[1:1 + nvb]
    o_ref = refs[1 + nvb]
    out = s_ref[...]
    parts = []
    for i in range(nvb):
        parts.append(aggs[i][0] + aggs[i][1])
    o_ref[...] = out + jnp.concatenate(parts, axis=1)


def _combine(skip, agg_parts):
    d = skip.shape[1]
    nvb = d // 128
    grid = NN // _ROW_BLK
    in_specs = [pl.BlockSpec((_ROW_BLK, d), lambda i: (i, 0))] + [
        pl.BlockSpec((2, _ROW_BLK, 128), lambda i: (0, i, 0))] * nvb
    return pl.pallas_call(
        functools.partial(_combine_body, nvb),
        grid=(grid,),
        in_specs=in_specs,
        out_specs=pl.BlockSpec((_ROW_BLK, d), lambda i: (i, 0)),
        out_shape=jax.ShapeDtypeStruct((NN, d), jnp.float32),
    )(skip, *agg_parts)


def _gap_body(c2_ref, b_ref, o_ref):
    gids = lax.broadcasted_iota(jnp.int32, (N_GRAPHS, NN), 0)
    mask = (b_ref[...] == gids).astype(jnp.float32)
    sums = jnp.dot(mask, c2_ref[...], preferred_element_type=jnp.float32)
    counts = jnp.sum(mask, axis=1, keepdims=True)
    o_ref[...] = sums / jnp.maximum(counts, 1.0)


def _gap(c2_pad, batch_pad):
    return pl.pallas_call(
        _gap_body,
        in_specs=[
            pl.BlockSpec((NN, c2_pad.shape[1]), lambda: (0, 0)),
            pl.BlockSpec((1, NN), lambda: (0, 0)),
        ],
        out_specs=pl.BlockSpec((N_GRAPHS, c2_pad.shape[1]), lambda: (0, 0)),
        out_shape=jax.ShapeDtypeStruct((N_GRAPHS, c2_pad.shape[1]), jnp.float32),
    )(c2_pad, batch_pad.reshape(1, -1))


_GDN = lax.GatherDimensionNumbers(
    offset_dims=(), collapsed_slice_dims=(0,), start_index_map=(0,))


def _lane_shift(x, s):
    idx = (lax.iota(jnp.int32, 16) + s) & 15
    return lax.gather(x, idx[:, None], dimension_numbers=_GDN,
                      slice_sizes=(1,),
                      mode=lax.GatherScatterMode.PROMISE_IN_BOUNDS)


def _exp0(x):
    return jnp.exp(jnp.maximum(x, -87.0))


def _butterfly_softmax(key, m, s):
    for sh in (1, 2, 4, 8):
        k2 = _lane_shift(key, sh)
        m2 = jnp.where(key == k2, _lane_shift(m, sh), NEG)
        s2 = jnp.where(key == k2, _lane_shift(s, sh), 0.0)
        mm = jnp.maximum(m, m2)
        s = s * _exp0(m - mm) + s2 * _exp0(m2 - mm)
        m = mm
    return m, s


def _worker_id():
    return lax.axis_index("c") * NS + lax.axis_index("s")


def _alpha_body(d, q_hbm, k_hbm, src_hbm, dst_hbm,
                alpha_hbm, mpart_hbm, spart_hbm,
                srcv0, srcv1, dstv0, dstv1, qbuf0, qbuf1, kbuf0, kbuf1,
                abuf0, abuf1, mloc, sloc,
                semi0, semi1, semg0, semg1, sema0, sema1):
    w = _worker_id()
    ch = _CHA(d)
    nch = EP // ch
    scale = 1.0 / float(d) ** 0.5
    srcv = (srcv0, srcv1)
    dstv = (dstv0, dstv1)
    qbuf = (qbuf0, qbuf1)
    kbuf = (kbuf0, kbuf1)
    abuf = (abuf0, abuf1)
    semi = (semi0, semi1)
    semg = (semg0, semg1)
    sema = (sema0, sema1)

    def init(i, _):
        mloc[pl.ds(i * 16, 16)] = jnp.full((16,), NEG, jnp.float32)
        sloc[pl.ds(i * 16, 16)] = jnp.zeros((16,), jnp.float32)
        return 0
    lax.fori_loop(0, NN // 16, init, 0)

    def issue_idx(ci, b):
        base = w * EP + ci * ch
        pltpu.async_copy(src_hbm.at[pl.ds(base, ch)], srcv[b], semi[b])
        pltpu.async_copy(dst_hbm.at[pl.ds(base, ch)], dstv[b], semi[b])

    def wait_idx(b):
        pltpu.make_async_copy(src_hbm.at[pl.ds(0, ch)], srcv[b], semi[b]).wait()
        pltpu.make_async_copy(dst_hbm.at[pl.ds(0, ch)], dstv[b], semi[b]).wait()

    def issue_gather(b):
        pltpu.async_copy(q_hbm.at[dstv[b]], qbuf[b], semg[b])
        pltpu.async_copy(k_hbm.at[srcv[b]], kbuf[b], semg[b])

    def wait_gather(b):
        pltpu.make_async_copy(q_hbm.at[dstv[b]], qbuf[b], semg[b]).wait()
        pltpu.make_async_copy(k_hbm.at[srcv[b]], kbuf[b], semg[b]).wait()

    def wait_alpha(b):
        pltpu.make_async_copy(abuf[b], alpha_hbm.at[pl.ds(0, ch)],
                              sema[b]).wait()

    def compute(ci, b):
        def grp(g, _):
            lanes = lax.iota(jnp.int32, 16) + g * 16

            def dot(jo, acc):
                for u in range(16):
                    jv = jnp.full((16,), jo * 16 + u, jnp.int32)
                    acc = acc + (plsc.load_gather(qbuf[b], [lanes, jv]) *
                                 plsc.load_gather(kbuf[b], [lanes, jv]))
                return acc
            alpha = lax.fori_loop(0, d // 16, dot,
                                  jnp.zeros((16,), jnp.float32)) * scale
            dst16 = dstv[b][pl.ds(g * 16, 16)]
            m, ss = _butterfly_softmax(dst16, alpha,
                                       jnp.ones((16,), jnp.float32))
            curm = plsc.load_gather(mloc, [dst16])
            curs = plsc.load_gather(sloc, [dst16])
            mm = jnp.maximum(curm, m)
            snew = curs * _exp0(curm - mm) + ss * _exp0(m - mm)
            plsc.store_scatter(mloc, [dst16], mm)
            plsc.store_scatter(sloc, [dst16], snew)
            abuf[b][pl.ds(g * 16, 16)] = alpha
            return 0
        lax.fori_loop(0, ch // 16, grp, 0)
        base = w * EP + ci * ch
        pltpu.async_copy(abuf[b], alpha_hbm.at[pl.ds(base, ch)], sema[b])

    issue_idx(0, 0)
    issue_idx(1, 1)
    wait_idx(0)
    issue_gather(0)
    wait_idx(1)
    issue_gather(1)

    def pair(i, _):
        c0 = 2 * i
        wait_gather(0)

        @pl.when(i > 0)
        def _():
            wait_alpha(0)
        compute(c0, 0)
        issue_idx(c0 + 2, 0)
        wait_gather(1)

        @pl.when(i > 0)
        def _():
            wait_alpha(1)
        compute(c0 + 1, 1)
        issue_idx(c0 + 3, 1)
        wait_idx(0)
        issue_gather(0)
        wait_idx(1)
        issue_gather(1)
        return 0
    lax.fori_loop(0, nch // 2 - 1, pair, 0)
    wait_gather(0)
    wait_alpha(0)
    compute(nch - 2, 0)
    wait_gather(1)
    wait_alpha(1)
    compute(nch - 1, 1)
    wait_alpha(0)
    wait_alpha(1)
    pltpu.sync_copy(mloc, mpart_hbm.at[w])
    pltpu.sync_copy(sloc, spart_hbm.at[w])


def _CHA(d):
    return 16384 // d


def _alpha_kernel(d):
    ch = _CHA(d)
    mesh = plsc.VectorSubcoreMesh(core_axis_name="c", subcore_axis_name="s")
    return pl.kernel(
        functools.partial(_alpha_body, d),
        out_type=(
            jax.ShapeDtypeStruct((EPAD,), jnp.float32),
            jax.ShapeDtypeStruct((NW, NN), jnp.float32),
            jax.ShapeDtypeStruct((NW, NN), jnp.float32),
        ),
        mesh=mesh,
        scratch_types=[
            pltpu.VMEM((ch,), jnp.int32),
            pltpu.VMEM((ch,), jnp.int32),
            pltpu.VMEM((ch,), jnp.int32),
            pltpu.VMEM((ch,), jnp.int32),
            pltpu.VMEM((ch, d), jnp.float32),
            pltpu.VMEM((ch, d), jnp.float32),
            pltpu.VMEM((ch, d), jnp.float32),
            pltpu.VMEM((ch, d), jnp.float32),
            pltpu.VMEM((ch,), jnp.float32),
            pltpu.VMEM((ch,), jnp.float32),
            pltpu.VMEM((NN,), jnp.float32),
            pltpu.VMEM((NN,), jnp.float32),
            pltpu.SemaphoreType.DMA,
            pltpu.SemaphoreType.DMA,
            pltpu.SemaphoreType.DMA,
            pltpu.SemaphoreType.DMA,
            pltpu.SemaphoreType.DMA,
            pltpu.SemaphoreType.DMA,
        ],
        compiler_params=_SC_PARAMS,
    )


def _merge_body(mpart_hbm, spart_hbm, mg_hbm, sg_hbm, blkm, blks, mgv, sgv):
    w = _worker_id()
    for t in range(NW):
        pltpu.sync_copy(mpart_hbm.at[pl.ds(t * NN + w * NP, NP)],
                        blkm.at[pl.ds(t * NP, NP)])
        pltpu.sync_copy(spart_hbm.at[pl.ds(t * NN + w * NP, NP)],
                        blks.at[pl.ds(t * NP, NP)])

    def col(i, _):
        m = jnp.full((16,), NEG, jnp.float32)
        for t in range(NW):
            m = jnp.maximum(m, blkm[pl.ds(t * NP + i * 16, 16)])
        s = jnp.zeros((16,), jnp.float32)
        for t in range(NW):
            mt = blkm[pl.ds(t * NP + i * 16, 16)]
            s = s + blks[pl.ds(t * NP + i * 16, 16)] * _exp0(mt - m)
        mgv[pl.ds(i * 16, 16)] = m
        sgv[pl.ds(i * 16, 16)] = s
        return 0
    lax.fori_loop(0, NP // 16, col, 0)
    pltpu.sync_copy(mgv, mg_hbm.at[pl.ds(w * NP, NP)])
    pltpu.sync_copy(sgv, sg_hbm.at[pl.ds(w * NP, NP)])


def _merge_kernel():
    mesh = plsc.VectorSubcoreMesh(core_axis_name="c", subcore_axis_name="s")
    return pl.kernel(
        _merge_body,
        out_type=(
            jax.ShapeDtypeStruct((NN,), jnp.float32),
            jax.ShapeDtypeStruct((NN,), jnp.float32),
        ),
        mesh=mesh,
        scratch_types=[
            pltpu.VMEM((NW * NP,), jnp.float32),
            pltpu.VMEM((NW * NP,), jnp.float32),
            pltpu.VMEM((NP,), jnp.float32),
            pltpu.VMEM((NP,), jnp.float32),
        ],
        compiler_params=_SC_PARAMS,
    )


CHC = 32
NCHC = EP // CHC


def _agg_body(nvb, *refs):
    v_blocks = refs[:nvb]
    src_hbm, dst_hbm, alpha_hbm, mg_hbm, sg_hbm = refs[nvb:nvb + 5]
    agg_outs = refs[nvb + 5:nvb + 5 + nvb]
    (srcv0, srcv1, dstv0, dstv1, dsts0, dsts1, abuf0, abuf1,
     vbuf0, vbuf1, sbuf0, sbuf1, zbuf, mv, sv, aggsp,
     semi0, semi1, semg0, semg1, sems0, sems1) = refs[nvb + 5 + nvb:]
    cid = lax.axis_index("c")
    sid = lax.axis_index("s")
    w = cid * NS + sid
    srcv = (srcv0, srcv1)
    dstv = (dstv0, dstv1)
    dsts = (dsts0, dsts1)
    abuf = (abuf0, abuf1)
    vbuf = (vbuf0, vbuf1)
    sbuf = (sbuf0, sbuf1)
    semi = (semi0, semi1)
    semg = (semg0, semg1)
    sems = (sems0, sems1)

    pltpu.sync_copy(mg_hbm, mv)
    pltpu.sync_copy(sg_hbm, sv)

    def zrow(r, _):
        for jj in range(8):
            zbuf[r, pl.ds(jj * 16, 16)] = jnp.zeros((16,), jnp.float32)
        return 0
    lax.fori_loop(0, 8, zrow, 0)

    def issue_idx(ci, b):
        base = w * EP + ci * CHC
        pltpu.async_copy(src_hbm.at[pl.ds(base, CHC)], srcv[b], semi[b])
        pltpu.async_copy(dst_hbm.at[pl.ds(base, CHC)], dstv[b], semi[b])
        pltpu.async_copy(alpha_hbm.at[pl.ds(base, CHC)], abuf[b], semi[b])

    def wait_idx(b):
        pltpu.make_async_copy(src_hbm.at[pl.ds(0, CHC)], srcv[b], semi[b]).wait()
        pltpu.make_async_copy(dst_hbm.at[pl.ds(0, CHC)], dstv[b], semi[b]).wait()
        pltpu.make_async_copy(alpha_hbm.at[pl.ds(0, CHC)], abuf[b], semi[b]).wait()

    def wait_scat(b):
        pltpu.make_async_copy(sbuf[b], aggsp.at[dsts[b]], sems[b]).wait()

    for blk in range(nvb):
        vb_hbm = v_blocks[blk]

        def issue_gather(b, _vb=vb_hbm):
            pltpu.async_copy(_vb.at[srcv[b]], vbuf[b], semg[b])

        def wait_gather(b, _vb=vb_hbm):
            pltpu.make_async_copy(_vb.at[srcv[b]], vbuf[b], semg[b]).wait()

        def process(ci, b):
            def grp(g, _):
                sl = pl.ds(g * 16, 16)
                dst16 = dstv[b][sl]
                a16 = abuf[b][sl]
                m16 = plsc.load_gather(mv, [dst16])
                s16 = plsc.load_gather(sv, [dst16])
                w16 = _exp0(a16 - m16) / (s16 + 1e-16)
                for u in range(16):
                    r = g * 16 + u
                    wv = jnp.full((16,), w16[u])
                    for jj in range(8):
                        cs = pl.ds(jj * 16, 16)
                        sbuf[b][r, cs] = vbuf[b][r, cs] * wv
                return 0
            lax.fori_loop(0, CHC // 16, grp, 0)
            for h in range(CHC // 16):
                hs = pl.ds(h * 16, 16)
                dsts[b][hs] = dstv[b][hs]
            pltpu.async_copy(sbuf[b], aggsp.at[dsts[b]], sems[b], add=True)

        def zsp(i, _):
            pltpu.sync_copy(zbuf, aggsp.at[pl.ds(sid * (NN // NS) + i * 8, 8)])
            return 0
        lax.fori_loop(0, NN // NS // 8, zsp, 0)
        plsc.subcore_barrier()

        issue_idx(0, 0)
        issue_idx(1, 1)
        wait_idx(0)
        issue_gather(0)
        wait_idx(1)
        issue_gather(1)

        def pair(i, _):
            c0 = 2 * i
            wait_gather(0)

            @pl.when(i > 0)
            def _():
                wait_scat(0)
            process(c0, 0)
            issue_idx(c0 + 2, 0)
            wait_gather(1)

            @pl.when(i > 0)
            def _():
                wait_scat(1)
            process(c0 + 1, 1)
            issue_idx(c0 + 3, 1)
            wait_idx(0)
            issue_gather(0)
            wait_idx(1)
            issue_gather(1)
            return 0
        lax.fori_loop(0, NCHC // 2 - 1, pair, 0)
        wait_gather(0)
        wait_scat(0)
        process(NCHC - 2, 0)
        wait_gather(1)
        wait_scat(1)
        process(NCHC - 1, 1)
        wait_scat(0)
        wait_scat(1)
        plsc.subcore_barrier()

        def dump(i, _):
            rowbase = sid * (NN // NS) + i * 128
            pltpu.sync_copy(aggsp.at[pl.ds(rowbase, 128)],
                            agg_outs[blk].at[cid, pl.ds(rowbase, 128)])
            return 0
        lax.fori_loop(0, NN // NS // 128, dump, 0)
        plsc.subcore_barrier()


def _agg_kernel(d):
    nvb = d // 128
    mesh = plsc.VectorSubcoreMesh(core_axis_name="c", subcore_axis_name="s")
    return pl.kernel(
        functools.partial(_agg_body, nvb),
        out_type=tuple(
            jax.ShapeDtypeStruct((2, NN, 128), jnp.float32)
            for _ in range(nvb)),
        mesh=mesh,
        scratch_types=[
            pltpu.VMEM((CHC,), jnp.int32),
            pltpu.VMEM((CHC,), jnp.int32),
            pltpu.VMEM((CHC,), jnp.int32),
            pltpu.VMEM((CHC,), jnp.int32),
            pltpu.VMEM((CHC,), jnp.int32),
            pltpu.VMEM((CHC,), jnp.int32),
            pltpu.VMEM((CHC,), jnp.float32),
            pltpu.VMEM((CHC,), jnp.float32),
            pltpu.VMEM((CHC, 128), jnp.float32),
            pltpu.VMEM((CHC, 128), jnp.float32),
            pltpu.VMEM((CHC, 128), jnp.float32),
            pltpu.VMEM((CHC, 128), jnp.float32),
            pltpu.VMEM((8, 128), jnp.float32),
            pltpu.VMEM((NN,), jnp.float32),
            pltpu.VMEM((NN,), jnp.float32),
            pltpu.VMEM_SHARED((NN, 128), jnp.float32),
            pltpu.SemaphoreType.DMA,
            pltpu.SemaphoreType.DMA,
            pltpu.SemaphoreType.DMA,
            pltpu.SemaphoreType.DMA,
            pltpu.SemaphoreType.DMA,
            pltpu.SemaphoreType.DMA,
        ],
        compiler_params=_SC_PARAMS,
    )


def _layer(xs, p, src_p, dst_p, act):
    d = p["Wq"].shape[1]
    w = jnp.concatenate([p["Wq"], p["Wk"], p["Wv"], p["Ws"]], axis=1)
    b = jnp.concatenate([p["bq"], p["bk"], p["bv"], p["bs"]])
    outs = _qkvs_matmul(xs, w, b, act)
    q, k, skip = outs[0], outs[1], outs[2]
    v_blocks = outs[3:]
    alpha, mpart, spart = _alpha_kernel(d)(q, k, src_p, dst_p)
    mg, sg = _merge_kernel()(mpart.reshape(-1), spart.reshape(-1))
    agg_parts = _agg_kernel(d)(*v_blocks, src_p, dst_p, alpha, mg, sg)
    if not isinstance(agg_parts, (list, tuple)):
        agg_parts = (agg_parts,)
    return _combine(skip, agg_parts)


def kernel(features, img_feat, edge_index, batch_index, params):
    pad_n = NN - N_NODES
    feat_p = jnp.pad(features, ((0, pad_n), (0, 0)))
    img_p = jnp.pad(img_feat, ((0, pad_n), (0, 0)))
    src_p = jnp.pad(edge_index[0], (0, EPAD - N_EDGES))
    dst_p = jnp.pad(edge_index[1], (0, EPAD - N_EDGES),
                    constant_values=N_NODES)
    batch_p = jnp.pad(batch_index, (0, pad_n), constant_values=N_GRAPHS)

    h1 = _layer((feat_p,), params["conv1"], src_p, dst_p, act=False)
    h2 = _layer((h1,), params["conv2"], src_p, dst_p, act=True)
    h3p = _layer((h2,), params["conv3"], src_p, dst_p, act=False)
    h4 = _layer((h3p,), params["conv4"], src_p, dst_p, act=True)
    img1p = _layer((img_p,), params["imgconv1"], src_p, dst_p, act=False)
    img2 = _layer((img1p,), params["imgconv2"], src_p, dst_p, act=True)
    img3p = _layer((img2,), params["imgconv3"], src_p, dst_p, act=False)
    img4 = _layer((img3p,), params["imgconv4"], src_p, dst_p, act=True)
    combine_p = _layer((h2, img2), params["neck"], src_p, dst_p, act=False)
    c2 = _layer((combine_p,), params["neck2"], src_p, dst_p, act=True)
    c3p = _layer((c2,), params["c3"], src_p, dst_p, act=False)
    c4 = _layer((c3p,), params["c4"], src_p, dst_p, act=True)
    hidden = _gap(c2, batch_p)
    return (h2[:N_NODES], img2[:N_NODES], c2[:N_NODES], h4[:N_NODES],
            img4[:N_NODES], c4[:N_NODES], hidden)
